# baseline (speedup 1.0000x reference)
import functools

import jax
import jax.numpy as jnp
from jax import lax
from jax.experimental import pallas as pl
from jax.experimental.pallas import tpu as pltpu
from jax.experimental.pallas import tpu_sc as plsc

B = 1024
CTX = 50
D = 64
V = 100000

NUM_CORES = 2
NUM_SUBCORES = 16
NW = NUM_CORES * NUM_SUBCORES
LANES = 16
ROUNDS = D // NW
GROUPS = 8
SUB = 128 // LANES


NBUF = 4
UNROLL = 5


def _pool_kernel(
    idx_hbm, table_hbm, out_hbm, idx_v, row_v, out_v, sem, s0, s1, s2, s3
):
    wid = lax.axis_index("s") * NUM_CORES + lax.axis_index("c")
    inv = jnp.float32(1.0 / CTX)
    isems = (s0, s1, s2, s3)

    def idx_fetch(g):
        return pltpu.async_copy(
            idx_hbm.at[:, pl.ds(g * 128, 128)], idx_v.at[g % NBUF], isems[g % NBUF]
        )

    for rnd in range(ROUNDS):
        f = wid + rnd * NW
        row_copy = pltpu.async_copy(table_hbm.at[f], row_v, sem)
        handles = {g: idx_fetch(g) for g in range(NBUF)}
        row_copy.wait()
        for g in range(GROUPS):
            handles.pop(g).wait()
            ib = idx_v.at[g % NBUF]

            def ctx_body(j5, accs):
                j = j5 * UNROLL
                for u in range(UNROLL):
                    accs = tuple(
                        accs[c]
                        + plsc.load_gather(
                            row_v, [ib[j + u, pl.ds(c * LANES, LANES)]]
                        )
                        for c in range(SUB)
                    )
                return accs

            accs = lax.fori_loop(
                0,
                CTX // UNROLL,
                ctx_body,
                tuple(jnp.zeros((LANES,), jnp.float32) for _ in range(SUB)),
            )
            for c in range(SUB):
                out_v[rnd, pl.ds(g * 128 + c * LANES, LANES)] = accs[c] * inv
            if g + NBUF < GROUPS:
                handles[g + NBUF] = idx_fetch(g + NBUF)
        pltpu.sync_copy(out_v.at[rnd], out_hbm.at[f])


def _pool_sc(idx_t, table_t):
    mesh = plsc.VectorSubcoreMesh(core_axis_name="c", subcore_axis_name="s")
    k = functools.partial(
        pl.kernel,
        mesh=mesh,
        out_type=jax.ShapeDtypeStruct((D, B), jnp.float32),
        scratch_types=[
            pltpu.VMEM((NBUF, CTX, 128), jnp.int32),
            pltpu.VMEM((V,), jnp.float32),
            pltpu.VMEM((ROUNDS, B), jnp.float32),
            pltpu.SemaphoreType.DMA,
            pltpu.SemaphoreType.DMA,
            pltpu.SemaphoreType.DMA,
            pltpu.SemaphoreType.DMA,
            pltpu.SemaphoreType.DMA,
        ],
        compiler_params=pltpu.CompilerParams(needs_layout_passes=False),
    )(_pool_kernel)
    return k(idx_t, table_t)


VB = 2048
GRID_V = (V + VB - 1) // VB


def _proj_kernel(xt_ref, wt_ref, b_ref, o_ref):
    ones_row = jnp.ones((1, B), dtype=jnp.float32)
    o_ref[...] = lax.dot_general(
        wt_ref[...],
        xt_ref[...],
        dimension_numbers=(((0,), (0,)), ((), ())),
        preferred_element_type=jnp.float32,
    ) + lax.dot_general(
        b_ref[...],
        ones_row,
        dimension_numbers=(((0,), (0,)), ((), ())),
        preferred_element_type=jnp.float32,
    )


def _project(pooled_t, Wt, b2d):
    out_t = pl.pallas_call(
        _proj_kernel,
        grid=(GRID_V,),
        in_specs=[
            pl.BlockSpec((D, B), lambda i: (0, 0)),
            pl.BlockSpec((D, VB), lambda i: (0, i)),
            pl.BlockSpec((1, VB), lambda i: (0, i)),
        ],
        out_specs=pl.BlockSpec((VB, B), lambda i: (i, 0)),
        out_shape=jax.ShapeDtypeStruct((V, B), jnp.float32),
    )(pooled_t, Wt, b2d)
    return out_t.T


def kernel(context_idxs, emb_table, W, b):
    pooled_t = _pool_sc(context_idxs.astype(jnp.int32).T, emb_table.T)
    return _project(
        pooled_t.astype(jnp.bfloat16), W.T.astype(jnp.bfloat16), b.reshape(1, V)
    )


# speedup vs baseline: 1.0080x; 1.0080x over previous
import functools

import jax
import jax.numpy as jnp
from jax import lax
from jax.experimental import pallas as pl
from jax.experimental.pallas import tpu as pltpu
from jax.experimental.pallas import tpu_sc as plsc

B = 1024
CTX = 50
D = 64
V = 100000

NUM_CORES = 2
NUM_SUBCORES = 16
NW = NUM_CORES * NUM_SUBCORES
LANES = 16
ROUNDS = D // NW
GROUPS = 8
SUB = 128 // LANES


NBUF = 4
UNROLL = 2


def _pool_kernel(
    idx_hbm, table_hbm, out_hbm, idx_v, row_v, out_v, sem, s0, s1, s2, s3
):
    wid = lax.axis_index("s") * NUM_CORES + lax.axis_index("c")
    inv = jnp.float32(1.0 / CTX)
    isems = (s0, s1, s2, s3)

    def idx_fetch(g):
        return pltpu.async_copy(
            idx_hbm.at[:, pl.ds(g * 128, 128)], idx_v.at[g % NBUF], isems[g % NBUF]
        )

    for rnd in range(ROUNDS):
        f = wid + rnd * NW
        row_copy = pltpu.async_copy(table_hbm.at[f], row_v, sem)
        handles = {g: idx_fetch(g) for g in range(NBUF)}
        row_copy.wait()
        for g in range(GROUPS):
            handles.pop(g).wait()
            ib = idx_v.at[g % NBUF]

            def ctx_body(j2, accs):
                j = j2 * UNROLL
                for u in range(UNROLL):
                    accs = tuple(
                        accs[c]
                        + plsc.load_gather(
                            row_v, [ib[j + u, pl.ds(c * LANES, LANES)]]
                        )
                        for c in range(SUB)
                    )
                return accs

            accs = lax.fori_loop(
                0,
                CTX // UNROLL,
                ctx_body,
                tuple(jnp.zeros((LANES,), jnp.float32) for _ in range(SUB)),
            )
            for c in range(SUB):
                out_v[rnd, pl.ds(g * 128 + c * LANES, LANES)] = accs[c] * inv
            if g + NBUF < GROUPS:
                handles[g + NBUF] = idx_fetch(g + NBUF)
        pltpu.sync_copy(out_v.at[rnd], out_hbm.at[f])


def _pool_sc(idx_t, table_t):
    mesh = plsc.VectorSubcoreMesh(core_axis_name="c", subcore_axis_name="s")
    k = functools.partial(
        pl.kernel,
        mesh=mesh,
        out_type=jax.ShapeDtypeStruct((D, B), jnp.float32),
        scratch_types=[
            pltpu.VMEM((NBUF, CTX, 128), jnp.int32),
            pltpu.VMEM((V,), jnp.float32),
            pltpu.VMEM((ROUNDS, B), jnp.float32),
            pltpu.SemaphoreType.DMA,
            pltpu.SemaphoreType.DMA,
            pltpu.SemaphoreType.DMA,
            pltpu.SemaphoreType.DMA,
            pltpu.SemaphoreType.DMA,
        ],
        compiler_params=pltpu.CompilerParams(needs_layout_passes=False),
    )(_pool_kernel)
    return k(idx_t, table_t)


VB = 2048
GRID_V = (V + VB - 1) // VB


def _proj_kernel(xt_ref, wt_ref, b_ref, o_ref):
    ones_row = jnp.ones((1, B), dtype=jnp.float32)
    o_ref[...] = lax.dot_general(
        wt_ref[...],
        xt_ref[...],
        dimension_numbers=(((0,), (0,)), ((), ())),
        preferred_element_type=jnp.float32,
    ) + lax.dot_general(
        b_ref[...],
        ones_row,
        dimension_numbers=(((0,), (0,)), ((), ())),
        preferred_element_type=jnp.float32,
    )


def _project(pooled_t, Wt, b2d):
    out_t = pl.pallas_call(
        _proj_kernel,
        grid=(GRID_V,),
        in_specs=[
            pl.BlockSpec((D, B), lambda i: (0, 0)),
            pl.BlockSpec((D, VB), lambda i: (0, i)),
            pl.BlockSpec((1, VB), lambda i: (0, i)),
        ],
        out_specs=pl.BlockSpec((VB, B), lambda i: (i, 0)),
        out_shape=jax.ShapeDtypeStruct((V, B), jnp.float32),
    )(pooled_t, Wt, b2d)
    return out_t.T


def kernel(context_idxs, emb_table, W, b):
    pooled_t = _pool_sc(context_idxs.astype(jnp.int32).T, emb_table.T)
    return _project(
        pooled_t.astype(jnp.bfloat16), W.T.astype(jnp.bfloat16), b.reshape(1, V)
    )
